# Initial kernel scaffold; baseline (speedup 1.0000x reference)
import functools

import jax
import jax.numpy as jnp
from jax import lax
from jax.experimental import pallas as pl
from jax.experimental.pallas import tpu as pltpu
from jax.experimental.pallas import tpu_sc as plsc

VOCAB = 100000
EMBED_DIM = 128
OUTPUT_DIM = 1024
BATCH = 16384
SEQ = 20

NC = 2
NS = 16
NW = NC * NS
BPW = BATCH // NW
CB = 4
G = CB * SEQ
NCHUNK = BPW // CB

_mesh = plsc.VectorSubcoreMesh(core_axis_name="c", subcore_axis_name="s")


@functools.partial(
    pl.kernel,
    out_type=jax.ShapeDtypeStruct((BATCH, EMBED_DIM), jnp.float32),
    mesh=_mesh,
    scratch_types=[
        pltpu.VMEM((NCHUNK, G), jnp.int32),
        pltpu.VMEM((NCHUNK, G), jnp.int32),
        pltpu.VMEM((G, EMBED_DIM), jnp.float32),
        pltpu.VMEM((G, EMBED_DIM), jnp.float32),
        pltpu.VMEM((BPW, EMBED_DIM), jnp.float32),
        pltpu.SemaphoreType.DMA,
        pltpu.SemaphoreType.DMA,
    ],
)
def _pool(text_hbm, sidx_hbm, table_hbm, out_hbm,
          idx_v, sidx_v, rows0, rows1, pooled_v, gsem0, gsem1):
    cid = lax.axis_index("c")
    sid = lax.axis_index("s")
    wid = sid * NC + cid

    pltpu.sync_copy(text_hbm.at[wid], idx_v)
    pltpu.sync_copy(sidx_hbm, sidx_v)

    pltpu.async_copy(table_hbm.at[idx_v.at[0]], rows0, gsem0)

    zero = jnp.zeros((16,), jnp.float32)

    def _zero_body(i, carry):
        for d in range(EMBED_DIM // 16):
            pooled_v[i, pl.ds(d * 16, 16)] = zero
        return carry

    lax.fori_loop(0, BPW, _zero_body, 0)

    def _chunk_body(k, carry):
        c0 = 2 * k
        c1 = c0 + 1
        pltpu.make_async_copy(table_hbm.at[idx_v.at[c0]], rows0, gsem0).wait()
        pltpu.async_copy(table_hbm.at[idx_v.at[c1]], rows1, gsem1)
        pltpu.sync_copy(rows0, pooled_v.at[sidx_v.at[c0]], add=True)
        pltpu.make_async_copy(table_hbm.at[idx_v.at[c1]], rows1, gsem1).wait()

        @pl.when(k < NCHUNK // 2 - 1)
        def _():
            pltpu.async_copy(table_hbm.at[idx_v.at[c0 + 2]], rows0, gsem0)

        pltpu.sync_copy(rows1, pooled_v.at[sidx_v.at[c1]], add=True)
        return carry

    lax.fori_loop(0, NCHUNK // 2, _chunk_body, 0)

    pltpu.sync_copy(pooled_v, out_hbm.at[pl.ds(wid * BPW, BPW)])


_BM = 256


def _mm_body(x_ref, w_ref, b_ref, o_ref):
    x = x_ref[...] * jnp.float32(1.0 / SEQ)
    acc = lax.dot_general(x, w_ref[...], (((1,), (1,)), ((), ())),
                          preferred_element_type=jnp.float32)
    o_ref[...] = acc + b_ref[...]


_matmul = pl.pallas_call(
    _mm_body,
    grid=(BATCH // _BM,),
    in_specs=[
        pl.BlockSpec((_BM, EMBED_DIM), lambda i: (i, 0)),
        pl.BlockSpec((OUTPUT_DIM, EMBED_DIM), lambda i: (0, 0)),
        pl.BlockSpec((1, OUTPUT_DIM), lambda i: (0, 0)),
    ],
    out_specs=pl.BlockSpec((_BM, OUTPUT_DIM), lambda i: (i, 0)),
    out_shape=jax.ShapeDtypeStruct((BATCH, OUTPUT_DIM), jnp.float32),
)


def kernel(text, embedding_table, fc_weight, fc_bias):
    text = text.astype(jnp.int32).reshape(NW, NCHUNK, G)
    sidx = (jnp.arange(NCHUNK * G, dtype=jnp.int32) // SEQ).reshape(NCHUNK, G)
    sums = _pool(text, sidx, embedding_table)
    return _matmul(sums, fc_weight, fc_bias.reshape(1, OUTPUT_DIM))


# speedup vs baseline: 5.7783x; 5.7783x over previous
import functools

import jax
import jax.numpy as jnp
from jax import lax
from jax.experimental import pallas as pl
from jax.experimental.pallas import tpu as pltpu
from jax.experimental.pallas import tpu_sc as plsc

VOCAB = 100000
EMBED_DIM = 128
OUTPUT_DIM = 1024
BATCH = 16384
SEQ = 20
LANES = 16
ND = EMBED_DIM // LANES

NC = 2
NS = 16
NW = NC * NS
BPW = BATCH // NW
CB = 4
G = CB * SEQ
NCHUNK = BPW // CB

_mesh = plsc.VectorSubcoreMesh(core_axis_name="c", subcore_axis_name="s")


@functools.partial(
    pl.kernel,
    out_type=jax.ShapeDtypeStruct((BATCH, EMBED_DIM), jnp.float32),
    mesh=_mesh,
    scratch_types=[
        pltpu.VMEM((NCHUNK, G), jnp.int32),
        pltpu.VMEM((G, EMBED_DIM), jnp.float32),
        pltpu.VMEM((G, EMBED_DIM), jnp.float32),
        pltpu.VMEM((BPW, EMBED_DIM), jnp.float32),
        pltpu.SemaphoreType.DMA,
        pltpu.SemaphoreType.DMA,
    ],
)
def _pool(text_hbm, table_hbm, out_hbm,
          idx_v, rows0, rows1, pooled_v, gsem0, gsem1):
    cid = lax.axis_index("c")
    sid = lax.axis_index("s")
    wid = sid * NC + cid

    pltpu.sync_copy(text_hbm.at[wid], idx_v)

    pltpu.async_copy(table_hbm.at[idx_v.at[0]], rows0, gsem0)
    pltpu.async_copy(table_hbm.at[idx_v.at[1]], rows1, gsem1)

    def _accumulate(buf, c):
        for q in range(CB):
            row = CB * c + q
            accs = [buf[SEQ * q, pl.ds(d * LANES, LANES)] for d in range(ND)]
            for j in range(1, SEQ):
                for d in range(ND):
                    accs[d] = accs[d] + buf[SEQ * q + j, pl.ds(d * LANES, LANES)]
            for d in range(ND):
                pooled_v[row, pl.ds(d * LANES, LANES)] = accs[d]

    def _chunk_body(k, carry):
        c0 = 2 * k
        c1 = c0 + 1
        pltpu.make_async_copy(table_hbm.at[idx_v.at[c0]], rows0, gsem0).wait()
        _accumulate(rows0, c0)

        @pl.when(k < NCHUNK // 2 - 1)
        def _():
            pltpu.async_copy(table_hbm.at[idx_v.at[c0 + 2]], rows0, gsem0)

        pltpu.make_async_copy(table_hbm.at[idx_v.at[c1]], rows1, gsem1).wait()
        _accumulate(rows1, c1)

        @pl.when(k < NCHUNK // 2 - 1)
        def _():
            pltpu.async_copy(table_hbm.at[idx_v.at[c1 + 2]], rows1, gsem1)

        return carry

    lax.fori_loop(0, NCHUNK // 2, _chunk_body, 0)

    pltpu.sync_copy(pooled_v, out_hbm.at[pl.ds(wid * BPW, BPW)])


_BM = 256


def _mm_body(x_ref, w_ref, b_ref, o_ref):
    x = x_ref[...] * jnp.float32(1.0 / SEQ)
    acc = lax.dot_general(x, w_ref[...], (((1,), (1,)), ((), ())),
                          preferred_element_type=jnp.float32)
    o_ref[...] = acc + b_ref[...]


_matmul = pl.pallas_call(
    _mm_body,
    grid=(BATCH // _BM,),
    in_specs=[
        pl.BlockSpec((_BM, EMBED_DIM), lambda i: (i, 0)),
        pl.BlockSpec((OUTPUT_DIM, EMBED_DIM), lambda i: (0, 0)),
        pl.BlockSpec((1, OUTPUT_DIM), lambda i: (0, 0)),
    ],
    out_specs=pl.BlockSpec((_BM, OUTPUT_DIM), lambda i: (i, 0)),
    out_shape=jax.ShapeDtypeStruct((BATCH, OUTPUT_DIM), jnp.float32),
)


def kernel(text, embedding_table, fc_weight, fc_bias):
    text = text.astype(jnp.int32).reshape(NW, NCHUNK, G)
    sums = _pool(text, embedding_table)
    return _matmul(sums, fc_weight, fc_bias.reshape(1, OUTPUT_DIM))
